# Initial kernel scaffold; baseline (speedup 1.0000x reference)
import jax
import jax.numpy as jnp
from jax.experimental import pallas as pl


def kernel(x, edge_index, W_l, W_r, att, bias, gamma, beta, alpha_gn):
    raise NotImplementedError("write your pallas kernel here")


# speedup vs baseline: 7.6945x; 7.6945x over previous
import functools

import jax
import jax.numpy as jnp
from jax import lax
from jax.experimental import pallas as pl
from jax.experimental.pallas import tpu as pltpu
from jax.experimental.pallas import tpu_sc as plsc

N = 10000
D = 128
C = 128
E = 320000
NEG_SLOPE = 0.2

NC = 2
NS = 16
NW = NC * NS
B = 128
EP = 331776
EW = EP // NW
CHUNKS = EW // B
CW = C + 16
NP = 10016
RT = NP // NS


def _mm_body(x_ref, wl_ref, wr_ref, xl_ref, xr_ref):
    xb = x_ref[...]
    xl_ref[...] = jnp.dot(xb, wl_ref[...], preferred_element_type=jnp.float32)
    xr_ref[...] = jnp.dot(xb, wr_ref[...], preferred_element_type=jnp.float32)


def _project(x, W_l, W_r):
    blk = 1000
    grid = N // blk
    return pl.pallas_call(
        _mm_body,
        grid=(grid,),
        in_specs=[
            pl.BlockSpec((blk, D), lambda i: (i, 0)),
            pl.BlockSpec((D, C), lambda i: (0, 0)),
            pl.BlockSpec((D, C), lambda i: (0, 0)),
        ],
        out_specs=[
            pl.BlockSpec((blk, C), lambda i: (i, 0)),
            pl.BlockSpec((blk, C), lambda i: (i, 0)),
        ],
        out_shape=[
            jax.ShapeDtypeStruct((N, C), jnp.float32),
            jax.ShapeDtypeStruct((N, C), jnp.float32),
        ],
    )(x, W_l, W_r)


def _logits_body(xlp_hbm, xr_hbm, src_hbm, dstg_hbm, att_hbm,
                 e_hbm, tmax_hbm,
                 att_v, src_v, dst_v, rows_l, rows_r, e_v, mx_v, ts_v,
                 sem1, sem2):
    c = lax.axis_index("c")
    s = lax.axis_index("s")
    wid = s * NC + c
    base = wid * EW

    pltpu.sync_copy(att_hbm, att_v)
    att_regs = [att_v[pl.ds(j * 16, 16)] for j in range(C // 16)]

    def chunk(t, mx):
        off = base + t * B
        pltpu.sync_copy(src_hbm.at[pl.ds(off, B)], src_v)
        pltpu.sync_copy(dstg_hbm.at[pl.ds(off, B)], dst_v)
        cp1 = pltpu.make_async_copy(xlp_hbm.at[src_v], rows_l, sem1)
        cp2 = pltpu.make_async_copy(xr_hbm.at[dst_v], rows_r, sem2)
        cp1.start()
        cp2.start()
        cp1.wait()
        cp2.wait()

        iota = lax.iota(jnp.int32, 16)

        def egroup(g, carry):
            for l in range(16):
                i = g * 16 + l
                acc = jnp.zeros((16,), jnp.float32)
                for j in range(C // 16):
                    m = (rows_l[i, pl.ds(j * 16, 16)]
                         + rows_r[i, pl.ds(j * 16, 16)])
                    m = jnp.maximum(m, NEG_SLOPE * m)
                    acc = acc + att_regs[j] * m
                ts_v[pl.ds(l * 16, 16)] = acc
            tot = jnp.zeros((16,), jnp.float32)
            iota16 = iota * 16
            for cc in range(16):
                tot = tot + plsc.load_gather(ts_v, [iota16 + cc])
            e_v[pl.ds(g * 16, 16)] = tot
            return carry

        lax.fori_loop(0, B // 16, egroup, 0)
        pltpu.sync_copy(e_v, e_hbm.at[pl.ds(off, B)])
        for k in range(B // 16):
            mx = jnp.maximum(mx, e_v[pl.ds(k * 16, 16)])
        return mx

    mx = lax.fori_loop(0, CHUNKS, chunk, jnp.full((16,), -1e30, jnp.float32))
    mx_v[...] = mx
    pltpu.sync_copy(mx_v, tmax_hbm.at[wid])


def _logits(xlp, xr, src, dstg, att):
    mesh = plsc.VectorSubcoreMesh(core_axis_name="c", subcore_axis_name="s",
                                  num_cores=NC, num_subcores=NS)
    return pl.kernel(
        _logits_body,
        out_type=(
            jax.ShapeDtypeStruct((EP,), jnp.float32),
            jax.ShapeDtypeStruct((NW, 16), jnp.float32),
        ),
        mesh=mesh,
        scratch_types=[
            pltpu.VMEM((C,), jnp.float32),
            pltpu.VMEM((B,), jnp.int32),
            pltpu.VMEM((B,), jnp.int32),
            pltpu.VMEM((B, CW), jnp.float32),
            pltpu.VMEM((B, C), jnp.float32),
            pltpu.VMEM((B,), jnp.float32),
            pltpu.VMEM((16,), jnp.float32),
            pltpu.VMEM((256,), jnp.float32),
            pltpu.SemaphoreType.DMA,
            pltpu.SemaphoreType.DMA,
        ],
        compiler_params=pltpu.CompilerParams(needs_layout_passes=False, use_tc_tiling_on_sc=False),
    )(xlp, xr, src, dstg, att)


def _aggregate_body(xlp_hbm, src_hbm, dsts_hbm, e_hbm, tmax_hbm, zeros_hbm,
                    accs_hbm,
                    src_v, dst_v, e_v, rows_v, tm_v, acc_sh, sem1):
    c = lax.axis_index("c")
    s = lax.axis_index("s")
    wid = s * NC + c
    base = wid * EW

    pltpu.sync_copy(zeros_hbm.at[pl.ds(s * RT, RT)],
                    acc_sh.at[pl.ds(s * RT, RT)])
    plsc.subcore_barrier()

    pltpu.sync_copy(tmax_hbm, tm_v)
    gmv = jnp.full((16,), -1e30, jnp.float32)
    for r in range(NW):
        gmv = jnp.maximum(gmv, tm_v[r])
    gmax = jnp.max(gmv)

    def chunk(t, carry):
        off = base + t * B
        pltpu.sync_copy(src_hbm.at[pl.ds(off, B)], src_v)
        pltpu.sync_copy(dsts_hbm.at[pl.ds(off, B)], dst_v)
        pltpu.sync_copy(e_hbm.at[pl.ds(off, B)], e_v)
        pltpu.make_async_copy(xlp_hbm.at[src_v], rows_v, sem1).start()
        for k in range(B // 16):
            e_v[pl.ds(k * 16, 16)] = jnp.exp(e_v[pl.ds(k * 16, 16)] - gmax)
        pltpu.make_async_copy(xlp_hbm.at[src_v], rows_v, sem1).wait()

        def edge(i, icarry):
            bw = plsc.load_gather(e_v, [jnp.full((16,), i, jnp.int32)])
            for j in range(CW // 16):
                rows_v[i, pl.ds(j * 16, 16)] = rows_v[i, pl.ds(j * 16, 16)] * bw
            return icarry

        lax.fori_loop(0, B, edge, 0)
        pltpu.sync_copy(rows_v, acc_sh.at[dst_v], add=True)
        return carry

    lax.fori_loop(0, CHUNKS, chunk, 0)
    plsc.subcore_barrier()
    pltpu.sync_copy(acc_sh.at[pl.ds(s * RT, RT)],
                    accs_hbm.at[c, pl.ds(s * RT, RT)])


def _aggregate(xlp, src, dsts, e, tmax, zeros_acc):
    mesh = plsc.VectorSubcoreMesh(core_axis_name="c", subcore_axis_name="s",
                                  num_cores=NC, num_subcores=NS)
    return pl.kernel(
        _aggregate_body,
        out_type=jax.ShapeDtypeStruct((NC, NP, CW), jnp.float32),
        mesh=mesh,
        scratch_types=[
            pltpu.VMEM((B,), jnp.int32),
            pltpu.VMEM((B,), jnp.int32),
            pltpu.VMEM((B,), jnp.float32),
            pltpu.VMEM((B, CW), jnp.float32),
            pltpu.VMEM((NW, 16), jnp.float32),
            pltpu.VMEM_SHARED((NP, CW), jnp.float32),
            pltpu.SemaphoreType.DMA,
        ],
        compiler_params=pltpu.CompilerParams(needs_layout_passes=False, use_tc_tiling_on_sc=False),
    )(xlp, src, dsts, e, tmax, zeros_acc)


def _norm_body(accs_ref, bias_ref, gamma_ref, beta_ref, agn_ref, out_ref):
    a = accs_ref[0] + accs_ref[1]
    col = lax.broadcasted_iota(jnp.int32, (NP, CW), 1)
    den_full = jnp.where(col == C, a, 0.0)
    den = jnp.sum(den_full, axis=1, keepdims=True)
    num = a[:N, :C]
    out0 = num / (den[:N] + 1e-16) + bias_ref[...]
    mean = jnp.mean(out0, axis=0, keepdims=True)
    out_c = out0 - agn_ref[...] * mean
    var = jnp.mean(out_c * out_c, axis=0, keepdims=True)
    out_ref[...] = gamma_ref[...] * out_c / jnp.sqrt(var + 1e-5) + beta_ref[...]


def _finalize(accs, bias, gamma, beta, alpha_gn):
    return pl.pallas_call(
        _norm_body,
        out_shape=jax.ShapeDtypeStruct((N, C), jnp.float32),
    )(accs, bias.reshape(1, C), gamma.reshape(1, C), beta.reshape(1, C),
      alpha_gn.reshape(1, C))


@jax.jit
def kernel(x, edge_index, W_l, W_r, att, bias, gamma, beta, alpha_gn):
    ei = edge_index.astype(jnp.int32)
    loop = jnp.arange(N, dtype=jnp.int32)
    pad = EP - (E + N)
    src = jnp.concatenate([ei[0], loop, jnp.zeros((pad,), jnp.int32)])
    dstg = jnp.concatenate([ei[1], loop, jnp.zeros((pad,), jnp.int32)])
    dsts = jnp.concatenate([ei[1], loop, jnp.full((pad,), N, jnp.int32)])

    xl, xr = _project(x, W_l, W_r)
    extra = jnp.concatenate(
        [jnp.ones((N, 1), jnp.float32), jnp.zeros((N, 15), jnp.float32)], axis=1)
    xlp = jnp.concatenate([xl, extra], axis=1)

    e, tmax = _logits(xlp, xr, src, dstg, att.reshape(C))
    zeros_acc = jnp.zeros((NP, CW), jnp.float32)
    accs = _aggregate(xlp, src, dsts, e, tmax, zeros_acc)
    return _finalize(accs, bias, gamma, beta, alpha_gn)
